# baseline (speedup 1.0000x reference)
import functools

import jax
import jax.numpy as jnp
from jax.experimental import pallas as pl


def _sl1_sum(x):
    ax = jnp.abs(x)
    return jnp.sum(jnp.where(ax < 1.0, 0.5 * x * x, ax - 0.5),
                   axis=(0, 1), keepdims=True)


def _sl1_elt(x):
    ax = jnp.abs(x)
    return jnp.where(ax < 1.0, 0.5 * x * x, ax - 0.5)


def _pair_body(predT_ref, tag_ref, w_ref, loss_ref, lossc_ref, *, N, M, C, B, I):
    g = pl.program_id(0)

    @pl.when(g == 0)
    def _init():
        loss_ref[...] = jnp.zeros((1, 1), jnp.float32)
        lossc_ref[...] = jnp.zeros((1, 1), jnp.float32)

    predT = predT_ref[0]
    tag = tag_ref[0]
    w = w_ref[0]

    p_row = [predT[d : d + 1, :] for d in range(3)]

    run_min = jnp.full((1, N), jnp.inf, dtype=jnp.float32)
    best = [jnp.zeros((1, N), dtype=jnp.float32) for _ in range(3)]
    tmp2_sum = jnp.zeros((1, 1), jnp.float32)

    num_tiles = M // C
    for jb in range(num_tiles):
        c0 = jb * C
        t_col = [tag[c0 : c0 + C, d : d + 1] for d in range(3)]

        d0 = t_col[0] - p_row[0]
        d2m = d0 * d0
        d1 = t_col[1] - p_row[1]
        d2m = d2m + d1 * d1
        dd = t_col[2] - p_row[2]
        d2m = d2m + dd * dd

        lane = jax.lax.broadcasted_iota(jnp.int32, (C, N), 1)
        srow = jax.lax.broadcasted_iota(jnp.int32, (C, N), 0)

        cmin = jnp.min(d2m, axis=1, keepdims=True)
        carg = jnp.min(jnp.where(d2m == cmin, lane, N), axis=1, keepdims=True)
        csel = lane == carg
        for d in range(3):
            pp = jnp.sum(jnp.where(csel, p_row[d], 0.0), axis=1, keepdims=True)
            tmp2_sum = tmp2_sum + _sl1_sum(t_col[d] - pp)

        rmin = jnp.min(d2m, axis=0, keepdims=True)
        rarg = jnp.min(jnp.where(d2m == rmin, srow, C), axis=0, keepdims=True)
        rsel = srow == rarg
        upd = rmin < run_min
        run_min = jnp.where(upd, rmin, run_min)
        for d in range(3):
            bt = jnp.sum(jnp.where(rsel, t_col[d], 0.0), axis=0, keepdims=True)
            best[d] = jnp.where(upd, bt, best[d])

    tmp1_sum = jnp.zeros((1, 1), jnp.float32)
    csum = jnp.zeros((1, 1), jnp.float32)
    for d in range(3):
        tmp1_sum = tmp1_sum + _sl1_sum(p_row[d] - best[d])
        cp = jnp.sum(p_row[d], axis=(0, 1), keepdims=True) / N
        ct = jnp.sum(tag[:, d : d + 1], axis=(0, 1), keepdims=True) / M
        csum = csum + _sl1_elt(cp - ct)

    pair = w * (tmp1_sum / (3.0 * N) + tmp2_sum / (3.0 * M))
    loss_ref[...] += pair / B
    lossc_ref[...] += csum / (B * 3.0)


def kernel(X_v, target_X_v, weights, device=0):
    B, I, N, D = X_v.shape
    M = target_X_v.shape[2]
    G = B * I

    predT = jnp.transpose(X_v.reshape(G, N, D), (0, 2, 1))
    tag = target_X_v.reshape(G, M, D)
    w = weights.reshape(G, 1, 1).astype(jnp.float32)

    C = 128

    body = functools.partial(_pair_body, N=N, M=M, C=C, B=B, I=I)
    loss, lossc = pl.pallas_call(
        body,
        grid=(G,),
        in_specs=[
            pl.BlockSpec((1, D, N), lambda g: (g, 0, 0)),
            pl.BlockSpec((1, M, D), lambda g: (g, 0, 0)),
            pl.BlockSpec((1, 1, 1), lambda g: (g, 0, 0)),
        ],
        out_specs=[
            pl.BlockSpec((1, 1), lambda g: (0, 0)),
            pl.BlockSpec((1, 1), lambda g: (0, 0)),
        ],
        out_shape=[
            jax.ShapeDtypeStruct((1, 1), jnp.float32),
            jax.ShapeDtypeStruct((1, 1), jnp.float32),
        ],
    )(predT, tag, w)

    return (loss[0, 0], lossc[0, 0])
